# Initial kernel scaffold; baseline (speedup 1.0000x reference)
import jax
import jax.numpy as jnp
from jax.experimental import pallas as pl


def kernel(tokens, lengths, embeddings, W_ih, b_ih, W_hh, b_hh, W0, b0, W1, b1):
    raise NotImplementedError("write your pallas kernel here")


# speedup vs baseline: 91.3972x; 91.3972x over previous
import functools

import jax
import jax.numpy as jnp
from jax import lax
from jax.experimental import pallas as pl
from jax.experimental.pallas import tpu as pltpu
from jax.experimental.pallas import tpu_sc as plsc

MODEL_DIM = 128
MLP_DIM = 256
NUM_CLASSES = 3
B = 16
L = 2048
N_ROWS = B * L
NUM_WORKERS = 32
ROWS_PER_WORKER = N_ROWS // NUM_WORKERS
GCHUNK = 128
CHUNK = 256


def _gather_body(idx_hbm, table_hbm, out_hbm, idx_v, rows_v, sem):
    wid = lax.axis_index("s") * 2 + lax.axis_index("c")
    base = wid * ROWS_PER_WORKER
    for c in range(ROWS_PER_WORKER // GCHUNK):
        off = base + c * GCHUNK
        pltpu.sync_copy(idx_hbm.at[pl.ds(off, GCHUNK)], idx_v)
        pltpu.async_copy(table_hbm.at[idx_v], rows_v, sem).wait()
        pltpu.sync_copy(rows_v, out_hbm.at[pl.ds(off, GCHUNK)])


@functools.cache
def _make_sc_gather():
    return pl.kernel(
        _gather_body,
        out_type=jax.ShapeDtypeStruct((N_ROWS, MODEL_DIM), jnp.float32),
        mesh=plsc.VectorSubcoreMesh(core_axis_name="c", subcore_axis_name="s",
                                    num_cores=2, num_subcores=16),
        scratch_types=[
            pltpu.VMEM((GCHUNK,), jnp.int32),
            pltpu.VMEM((GCHUNK, MODEL_DIM), jnp.float32),
            pltpu.SemaphoreType.DMA,
        ],
    )


def _sc_gather(idx, table):
    return _make_sc_gather()(idx, table)


def _scan_body(x_ref, wih_ref, whh_ref, bias_ref, act_ref, w0_ref, b0_ref,
               w1_ref, b1_ref, out_ref, h_ref, pre_ref):
    g = pl.program_id(0)
    ng = pl.num_programs(0)

    @pl.when(g == 0)
    def _init():
        h_ref[...] = jnp.zeros_like(h_ref)

    pre_ref[...] = jnp.dot(x_ref[...], wih_ref[...],
                           preferred_element_type=jnp.float32) + bias_ref[...]
    whh = whh_ref[...]
    act = act_ref[...]
    t0 = g * CHUNK

    def body(j, h):
        x = pre_ref[pl.ds(j * B, B), :]
        h_new = jnp.tanh(x + jnp.dot(h, whh,
                                     preferred_element_type=jnp.float32))
        return jnp.where(act <= t0 + j, h_new, h)

    h = lax.fori_loop(0, CHUNK, body, h_ref[...])
    h_ref[...] = h

    @pl.when(g == ng - 1)
    def _mlp():
        hf = h_ref[...]
        h1 = jnp.maximum(
            jnp.dot(hf, w0_ref[...], preferred_element_type=jnp.float32)
            + b0_ref[...], 0.0)
        h2 = jnp.maximum(
            jnp.dot(h1, w1_ref[...], preferred_element_type=jnp.float32)
            + b1_ref[...], 0.0)
        lane = lax.broadcasted_iota(jnp.int32, (B, MODEL_DIM), 1)
        logits = jnp.where(lane < NUM_CLASSES, h2, -1e30)
        m = jnp.max(logits, axis=-1, keepdims=True)
        s = logits - m
        out_ref[...] = s - jnp.log(jnp.sum(jnp.exp(s), axis=-1,
                                           keepdims=True))


_scan_call = pl.pallas_call(
    _scan_body,
    grid=(L // CHUNK,),
    in_specs=[
        pl.BlockSpec((CHUNK * B, MODEL_DIM), lambda g: (g, 0)),
        pl.BlockSpec((MODEL_DIM, MODEL_DIM), lambda g: (0, 0)),
        pl.BlockSpec((MODEL_DIM, MODEL_DIM), lambda g: (0, 0)),
        pl.BlockSpec((1, MODEL_DIM), lambda g: (0, 0)),
        pl.BlockSpec((B, MODEL_DIM), lambda g: (0, 0)),
        pl.BlockSpec((MODEL_DIM, MLP_DIM), lambda g: (0, 0)),
        pl.BlockSpec((1, MLP_DIM), lambda g: (0, 0)),
        pl.BlockSpec((MLP_DIM, MODEL_DIM), lambda g: (0, 0)),
        pl.BlockSpec((1, MODEL_DIM), lambda g: (0, 0)),
    ],
    out_specs=pl.BlockSpec((B, MODEL_DIM), lambda g: (0, 0)),
    out_shape=jax.ShapeDtypeStruct((B, MODEL_DIM), jnp.float32),
    scratch_shapes=[pltpu.VMEM((B, MODEL_DIM), jnp.float32),
                    pltpu.VMEM((CHUNK * B, MODEL_DIM), jnp.float32)],
)


@jax.jit
def kernel(tokens, lengths, embeddings, W_ih, b_ih, W_hh, b_hh, W0, b0,
           W1, b1):
    idx = jnp.flip(tokens, axis=1).T.reshape(-1)
    x = _sc_gather(idx, embeddings)

    wih_t = W_ih.T
    whh_t = W_hh.T
    bias = (b_ih + b_hh)[None, :]
    act = jnp.broadcast_to((L - lengths)[:, None], (B, MODEL_DIM))
    act = act.astype(jnp.int32)
    w0_t = W0.T
    b0r = b0[None, :]
    w1p = jnp.zeros((MLP_DIM, MODEL_DIM), W1.dtype).at[:, :NUM_CLASSES].set(
        W1.T)
    b1p = jnp.zeros((1, MODEL_DIM), b1.dtype).at[0, :NUM_CLASSES].set(b1)

    y = _scan_call(x, wih_t, whh_t, bias, act, w0_t, b0r, w1p, b1p)
    return y[:, :NUM_CLASSES]


# speedup vs baseline: 100.9310x; 1.1043x over previous
import functools

import jax
import jax.numpy as jnp
from jax import lax
from jax.experimental import pallas as pl
from jax.experimental.pallas import tpu as pltpu
from jax.experimental.pallas import tpu_sc as plsc

MODEL_DIM = 128
MLP_DIM = 256
NUM_CLASSES = 3
B = 16
L = 2048
N_ROWS = B * L
NUM_WORKERS = 32
ROWS_PER_WORKER = N_ROWS // NUM_WORKERS
GCHUNK = 128
CHUNK = 256


def _gather_body(idx_hbm, table_hbm, out_hbm, idx_v, rows_v, sem):
    wid = lax.axis_index("s") * 2 + lax.axis_index("c")
    base = wid * ROWS_PER_WORKER
    for c in range(ROWS_PER_WORKER // GCHUNK):
        off = base + c * GCHUNK
        pltpu.sync_copy(idx_hbm.at[pl.ds(off, GCHUNK)], idx_v)
        pltpu.async_copy(table_hbm.at[idx_v], rows_v, sem).wait()
        pltpu.sync_copy(rows_v, out_hbm.at[pl.ds(off, GCHUNK)])


@functools.cache
def _make_sc_gather():
    return pl.kernel(
        _gather_body,
        out_type=jax.ShapeDtypeStruct((N_ROWS, MODEL_DIM), jnp.float32),
        mesh=plsc.VectorSubcoreMesh(core_axis_name="c", subcore_axis_name="s",
                                    num_cores=2, num_subcores=16),
        scratch_types=[
            pltpu.VMEM((GCHUNK,), jnp.int32),
            pltpu.VMEM((GCHUNK, MODEL_DIM), jnp.float32),
            pltpu.SemaphoreType.DMA,
        ],
    )


def _sc_gather(idx, table):
    return _make_sc_gather()(idx, table)


def _scan_body(x_ref, wih_ref, whh_ref, bias_ref, act_ref, w0_ref, b0_ref,
               w1_ref, b1_ref, out_ref, h_ref, pre_ref):
    g = pl.program_id(0)
    ng = pl.num_programs(0)

    @pl.when(g == 0)
    def _init():
        h_ref[...] = jnp.zeros_like(h_ref)

    pre_ref[...] = jnp.dot(x_ref[...], wih_ref[...],
                           preferred_element_type=jnp.float32) + bias_ref[...]
    whh = whh_ref[...]
    act = act_ref[...]
    t0 = g * CHUNK

    def body(j, h):
        x = pre_ref[pl.ds(j * B, B), :]
        h_new = jnp.tanh(x + jnp.dot(h, whh,
                                     preferred_element_type=jnp.float32))
        return jnp.where(act <= t0 + j, h_new, h)

    h = lax.fori_loop(0, CHUNK, body, h_ref[...], unroll=8)
    h_ref[...] = h

    @pl.when(g == ng - 1)
    def _mlp():
        hf = h_ref[...]
        h1 = jnp.maximum(
            jnp.dot(hf, w0_ref[...], preferred_element_type=jnp.float32)
            + b0_ref[...], 0.0)
        h2 = jnp.maximum(
            jnp.dot(h1, w1_ref[...], preferred_element_type=jnp.float32)
            + b1_ref[...], 0.0)
        lane = lax.broadcasted_iota(jnp.int32, (B, MODEL_DIM), 1)
        logits = jnp.where(lane < NUM_CLASSES, h2, -1e30)
        m = jnp.max(logits, axis=-1, keepdims=True)
        s = logits - m
        out_ref[...] = s - jnp.log(jnp.sum(jnp.exp(s), axis=-1,
                                           keepdims=True))


_scan_call = pl.pallas_call(
    _scan_body,
    grid=(L // CHUNK,),
    in_specs=[
        pl.BlockSpec((CHUNK * B, MODEL_DIM), lambda g: (g, 0)),
        pl.BlockSpec((MODEL_DIM, MODEL_DIM), lambda g: (0, 0)),
        pl.BlockSpec((MODEL_DIM, MODEL_DIM), lambda g: (0, 0)),
        pl.BlockSpec((1, MODEL_DIM), lambda g: (0, 0)),
        pl.BlockSpec((B, MODEL_DIM), lambda g: (0, 0)),
        pl.BlockSpec((MODEL_DIM, MLP_DIM), lambda g: (0, 0)),
        pl.BlockSpec((1, MLP_DIM), lambda g: (0, 0)),
        pl.BlockSpec((MLP_DIM, MODEL_DIM), lambda g: (0, 0)),
        pl.BlockSpec((1, MODEL_DIM), lambda g: (0, 0)),
    ],
    out_specs=pl.BlockSpec((B, MODEL_DIM), lambda g: (0, 0)),
    out_shape=jax.ShapeDtypeStruct((B, MODEL_DIM), jnp.float32),
    scratch_shapes=[pltpu.VMEM((B, MODEL_DIM), jnp.float32),
                    pltpu.VMEM((CHUNK * B, MODEL_DIM), jnp.float32)],
)


@jax.jit
def kernel(tokens, lengths, embeddings, W_ih, b_ih, W_hh, b_hh, W0, b0,
           W1, b1):
    idx = jnp.flip(tokens, axis=1).T.reshape(-1)
    x = _sc_gather(idx, embeddings)

    wih_t = W_ih.T
    whh_t = W_hh.T
    bias = (b_ih + b_hh)[None, :]
    act = jnp.broadcast_to((L - lengths)[:, None], (B, MODEL_DIM))
    act = act.astype(jnp.int32)
    w0_t = W0.T
    b0r = b0[None, :]
    w1p = jnp.zeros((MLP_DIM, MODEL_DIM), W1.dtype).at[:, :NUM_CLASSES].set(
        W1.T)
    b1p = jnp.zeros((1, MODEL_DIM), b1.dtype).at[0, :NUM_CLASSES].set(b1)

    y = _scan_call(x, wih_t, whh_t, bias, act, w0_t, b0r, w1p, b1p)
    return y[:, :NUM_CLASSES]


# speedup vs baseline: 104.9701x; 1.0400x over previous
import functools

import jax
import jax.numpy as jnp
from jax import lax
from jax.experimental import pallas as pl
from jax.experimental.pallas import tpu as pltpu
from jax.experimental.pallas import tpu_sc as plsc

MODEL_DIM = 128
MLP_DIM = 256
NUM_CLASSES = 3
B = 16
L = 2048
NUM_WORKERS = 32
HALF_ROWS = B * L // 2
ROWS_PER_WORKER = HALF_ROWS // NUM_WORKERS
GCHUNK = 128
NCHUNK = ROWS_PER_WORKER // GCHUNK
CHUNK = 256
HALF_STEPS = L // 2


def _gather_body(idx_hbm, table_hbm, out_hbm, idx_a, idx_b, rows_a, rows_b,
                 gsem_a, gsem_b, wsem_a, wsem_b):
    wid = lax.axis_index("s") * 2 + lax.axis_index("c")
    base = wid * ROWS_PER_WORKER
    idx = [idx_a, idx_b]
    rows = [rows_a, rows_b]
    gsem = [gsem_a, gsem_b]
    wsem = [wsem_a, wsem_b]
    gcopy = [None, None]
    wcopy = [None, None]

    pltpu.sync_copy(idx_hbm.at[pl.ds(base, GCHUNK)], idx_a)
    gcopy[0] = pltpu.async_copy(table_hbm.at[idx_a], rows_a, gsem_a)
    for c in range(NCHUNK):
        cur = c % 2
        nxt = 1 - cur
        if c + 1 < NCHUNK:
            off_n = base + (c + 1) * GCHUNK
            pltpu.sync_copy(idx_hbm.at[pl.ds(off_n, GCHUNK)], idx[nxt])
            if wcopy[nxt] is not None:
                wcopy[nxt].wait()
            gcopy[nxt] = pltpu.async_copy(table_hbm.at[idx[nxt]], rows[nxt],
                                          gsem[nxt])
        gcopy[cur].wait()
        wcopy[cur] = pltpu.async_copy(
            rows[cur], out_hbm.at[pl.ds(base + c * GCHUNK, GCHUNK)],
            wsem[cur])
    for w in wcopy:
        if w is not None:
            w.wait()


@functools.cache
def _make_sc_gather():
    return pl.kernel(
        _gather_body,
        out_type=jax.ShapeDtypeStruct((HALF_ROWS, MODEL_DIM), jnp.float32),
        mesh=plsc.VectorSubcoreMesh(core_axis_name="c", subcore_axis_name="s",
                                    num_cores=2, num_subcores=16),
        scratch_types=[
            pltpu.VMEM((GCHUNK,), jnp.int32),
            pltpu.VMEM((GCHUNK,), jnp.int32),
            pltpu.VMEM((GCHUNK, MODEL_DIM), jnp.float32),
            pltpu.VMEM((GCHUNK, MODEL_DIM), jnp.float32),
            pltpu.SemaphoreType.DMA,
            pltpu.SemaphoreType.DMA,
            pltpu.SemaphoreType.DMA,
            pltpu.SemaphoreType.DMA,
        ],
    )


def _recurrence(pre_ref, whh, act, t_base, h0):
    def body(j, h):
        x = pre_ref[pl.ds(j * B, B), :]
        h_new = jnp.tanh(x + jnp.dot(h, whh,
                                     preferred_element_type=jnp.float32))
        return jnp.where(act <= t_base + j, h_new, h)

    return lax.fori_loop(0, CHUNK, body, h0, unroll=8)


def _scan_a_body(x_ref, wih_ref, whh_ref, bias_ref, act_ref, out_ref,
                 h_ref, pre_ref):
    g = pl.program_id(0)
    ng = pl.num_programs(0)

    @pl.when(g == 0)
    def _init():
        h_ref[...] = jnp.zeros_like(h_ref)

    pre_ref[...] = jnp.dot(x_ref[...], wih_ref[...],
                           preferred_element_type=jnp.float32) + bias_ref[...]
    h = _recurrence(pre_ref, whh_ref[...], act_ref[...], g * CHUNK,
                    h_ref[...])
    h_ref[...] = h

    @pl.when(g == ng - 1)
    def _emit():
        out_ref[...] = h


def _scan_b_body(x_ref, wih_ref, whh_ref, bias_ref, act_ref, h_in_ref,
                 w0_ref, b0_ref, w1_ref, b1_ref, out_ref, h_ref, pre_ref):
    g = pl.program_id(0)
    ng = pl.num_programs(0)

    @pl.when(g == 0)
    def _init():
        h_ref[...] = h_in_ref[...]

    pre_ref[...] = jnp.dot(x_ref[...], wih_ref[...],
                           preferred_element_type=jnp.float32) + bias_ref[...]
    h = _recurrence(pre_ref, whh_ref[...], act_ref[...],
                    HALF_STEPS + g * CHUNK, h_ref[...])
    h_ref[...] = h

    @pl.when(g == ng - 1)
    def _mlp():
        h1 = jnp.maximum(
            jnp.dot(h, w0_ref[...], preferred_element_type=jnp.float32)
            + b0_ref[...], 0.0)
        h2 = jnp.maximum(
            jnp.dot(h1, w1_ref[...], preferred_element_type=jnp.float32)
            + b1_ref[...], 0.0)
        lane = lax.broadcasted_iota(jnp.int32, (B, MODEL_DIM), 1)
        logits = jnp.where(lane < NUM_CLASSES, h2, -1e30)
        m = jnp.max(logits, axis=-1, keepdims=True)
        s = logits - m
        out_ref[...] = s - jnp.log(jnp.sum(jnp.exp(s), axis=-1,
                                           keepdims=True))


def _full(shape):
    return pl.BlockSpec(shape, lambda g: tuple(0 for _ in shape))


_COMMON_SPECS = [
    pl.BlockSpec((CHUNK * B, MODEL_DIM), lambda g: (g, 0)),
    _full((MODEL_DIM, MODEL_DIM)),
    _full((MODEL_DIM, MODEL_DIM)),
    _full((1, MODEL_DIM)),
    _full((B, MODEL_DIM)),
]

_SCRATCH = [pltpu.VMEM((B, MODEL_DIM), jnp.float32),
            pltpu.VMEM((CHUNK * B, MODEL_DIM), jnp.float32)]

_scan_a_call = pl.pallas_call(
    _scan_a_body,
    grid=(HALF_STEPS // CHUNK,),
    in_specs=_COMMON_SPECS,
    out_specs=_full((B, MODEL_DIM)),
    out_shape=jax.ShapeDtypeStruct((B, MODEL_DIM), jnp.float32),
    scratch_shapes=_SCRATCH,
)

_scan_b_call = pl.pallas_call(
    _scan_b_body,
    grid=(HALF_STEPS // CHUNK,),
    in_specs=_COMMON_SPECS + [
        _full((B, MODEL_DIM)),
        _full((MODEL_DIM, MLP_DIM)),
        _full((1, MLP_DIM)),
        _full((MLP_DIM, MODEL_DIM)),
        _full((1, MODEL_DIM)),
    ],
    out_specs=_full((B, MODEL_DIM)),
    out_shape=jax.ShapeDtypeStruct((B, MODEL_DIM), jnp.float32),
    scratch_shapes=_SCRATCH,
)


@jax.jit
def kernel(tokens, lengths, embeddings, W_ih, b_ih, W_hh, b_hh, W0, b0,
           W1, b1):
    idx = jnp.flip(tokens, axis=1).T.reshape(-1)
    gather = _make_sc_gather()
    x0 = gather(idx[:HALF_ROWS], embeddings)
    x1 = gather(idx[HALF_ROWS:], embeddings)

    wih_t = W_ih.T
    whh_t = W_hh.T
    bias = (b_ih + b_hh)[None, :]
    act = jnp.broadcast_to((L - lengths)[:, None], (B, MODEL_DIM))
    act = act.astype(jnp.int32)
    w0_t = W0.T
    b0r = b0[None, :]
    w1p = jnp.zeros((MLP_DIM, MODEL_DIM), W1.dtype).at[:, :NUM_CLASSES].set(
        W1.T)
    b1p = jnp.zeros((1, MODEL_DIM), b1.dtype).at[0, :NUM_CLASSES].set(b1)

    h_mid = _scan_a_call(x0, wih_t, whh_t, bias, act)
    y = _scan_b_call(x1, wih_t, whh_t, bias, act, h_mid, w0_t, b0r, w1p,
                     b1p)
    return y[:, :NUM_CLASSES]


# speedup vs baseline: 105.3603x; 1.0037x over previous
import functools

import jax
import jax.numpy as jnp
from jax import lax
from jax.experimental import pallas as pl
from jax.experimental.pallas import tpu as pltpu
from jax.experimental.pallas import tpu_sc as plsc

MODEL_DIM = 128
MLP_DIM = 256
NUM_CLASSES = 3
B = 16
L = 2048
NUM_WORKERS = 32
HALF_ROWS = B * L // 2
ROWS_PER_WORKER = HALF_ROWS // NUM_WORKERS
GCHUNK = 128
NCHUNK = ROWS_PER_WORKER // GCHUNK
CHUNK = 512
BLK = 8
HALF_STEPS = L // 2


def _gather_body(idx_hbm, table_hbm, out_hbm, idx_a, idx_b, rows_a, rows_b,
                 gsem_a, gsem_b, wsem_a, wsem_b):
    wid = lax.axis_index("s") * 2 + lax.axis_index("c")
    base = wid * ROWS_PER_WORKER
    idx = [idx_a, idx_b]
    rows = [rows_a, rows_b]
    gsem = [gsem_a, gsem_b]
    wsem = [wsem_a, wsem_b]
    gcopy = [None, None]
    wcopy = [None, None]

    pltpu.sync_copy(idx_hbm.at[pl.ds(base, GCHUNK)], idx_a)
    gcopy[0] = pltpu.async_copy(table_hbm.at[idx_a], rows_a, gsem_a)
    for c in range(NCHUNK):
        cur = c % 2
        nxt = 1 - cur
        if c + 1 < NCHUNK:
            off_n = base + (c + 1) * GCHUNK
            pltpu.sync_copy(idx_hbm.at[pl.ds(off_n, GCHUNK)], idx[nxt])
            if wcopy[nxt] is not None:
                wcopy[nxt].wait()
            gcopy[nxt] = pltpu.async_copy(table_hbm.at[idx[nxt]], rows[nxt],
                                          gsem[nxt])
        gcopy[cur].wait()
        wcopy[cur] = pltpu.async_copy(
            rows[cur], out_hbm.at[pl.ds(base + c * GCHUNK, GCHUNK)],
            wsem[cur])
    for w in wcopy:
        if w is not None:
            w.wait()


@functools.cache
def _make_sc_gather():
    return pl.kernel(
        _gather_body,
        out_type=jax.ShapeDtypeStruct((HALF_ROWS, MODEL_DIM), jnp.float32),
        mesh=plsc.VectorSubcoreMesh(core_axis_name="c", subcore_axis_name="s",
                                    num_cores=2, num_subcores=16),
        scratch_types=[
            pltpu.VMEM((GCHUNK,), jnp.int32),
            pltpu.VMEM((GCHUNK,), jnp.int32),
            pltpu.VMEM((GCHUNK, MODEL_DIM), jnp.float32),
            pltpu.VMEM((GCHUNK, MODEL_DIM), jnp.float32),
            pltpu.SemaphoreType.DMA,
            pltpu.SemaphoreType.DMA,
            pltpu.SemaphoreType.DMA,
            pltpu.SemaphoreType.DMA,
        ],
    )


def _scan_chunk(x_ref, wih_ref, whh_ref, bias_ref, act_ref, m_ref, h_ref,
                pre_ref, t_base):
    @pl.when(m_ref[0] < t_base + CHUNK)
    def _active():
        pre_ref[...] = jnp.dot(
            x_ref[...], wih_ref[...],
            preferred_element_type=jnp.float32) + bias_ref[...]
        whh = whh_ref[...]
        act = act_ref[...]
        nblk0 = jnp.clip((m_ref[0] - t_base) // BLK, 0, CHUNK // BLK)

        def blk(kb, h):
            for jj in range(BLK):
                j = kb * BLK + jj
                x = pre_ref[pl.ds(j * B, B), :]
                h_new = jnp.tanh(
                    x + jnp.dot(h, whh, preferred_element_type=jnp.float32))
                h = jnp.where(act <= t_base + j, h_new, h)
            return h

        h_ref[...] = lax.fori_loop(nblk0, CHUNK // BLK, blk, h_ref[...])


def _scan_a_body(x_ref, wih_ref, whh_ref, bias_ref, act_ref, m_ref, out_ref,
                 h_ref, pre_ref):
    g = pl.program_id(0)
    ng = pl.num_programs(0)

    @pl.when(g == 0)
    def _init():
        h_ref[...] = jnp.zeros_like(h_ref)

    _scan_chunk(x_ref, wih_ref, whh_ref, bias_ref, act_ref, m_ref, h_ref,
                pre_ref, g * CHUNK)

    @pl.when(g == ng - 1)
    def _emit():
        out_ref[...] = h_ref[...]


def _scan_b_body(x_ref, wih_ref, whh_ref, bias_ref, act_ref, m_ref, h_in_ref,
                 w0_ref, b0_ref, w1_ref, b1_ref, out_ref, h_ref, pre_ref):
    g = pl.program_id(0)
    ng = pl.num_programs(0)

    @pl.when(g == 0)
    def _init():
        h_ref[...] = h_in_ref[...]

    _scan_chunk(x_ref, wih_ref, whh_ref, bias_ref, act_ref, m_ref, h_ref,
                pre_ref, HALF_STEPS + g * CHUNK)

    @pl.when(g == ng - 1)
    def _mlp():
        h1 = jnp.maximum(
            jnp.dot(h_ref[...], w0_ref[...],
                    preferred_element_type=jnp.float32)
            + b0_ref[...], 0.0)
        h2 = jnp.maximum(
            jnp.dot(h1, w1_ref[...], preferred_element_type=jnp.float32)
            + b1_ref[...], 0.0)
        lane = lax.broadcasted_iota(jnp.int32, (B, MODEL_DIM), 1)
        logits = jnp.where(lane < NUM_CLASSES, h2, -1e30)
        m = jnp.max(logits, axis=-1, keepdims=True)
        s = logits - m
        out_ref[...] = s - jnp.log(jnp.sum(jnp.exp(s), axis=-1,
                                           keepdims=True))


def _full(shape):
    return pl.BlockSpec(shape, lambda g: tuple(0 for _ in shape))


_COMMON_SPECS = [
    pl.BlockSpec((CHUNK * B, MODEL_DIM), lambda g: (g, 0)),
    _full((MODEL_DIM, MODEL_DIM)),
    _full((MODEL_DIM, MODEL_DIM)),
    _full((1, MODEL_DIM)),
    _full((B, MODEL_DIM)),
    pl.BlockSpec(memory_space=pltpu.SMEM),
]

_SCRATCH = [pltpu.VMEM((B, MODEL_DIM), jnp.float32),
            pltpu.VMEM((CHUNK * B, MODEL_DIM), jnp.float32)]

_scan_a_call = pl.pallas_call(
    _scan_a_body,
    grid=(HALF_STEPS // CHUNK,),
    in_specs=_COMMON_SPECS,
    out_specs=_full((B, MODEL_DIM)),
    out_shape=jax.ShapeDtypeStruct((B, MODEL_DIM), jnp.float32),
    scratch_shapes=_SCRATCH,
)

_scan_b_call = pl.pallas_call(
    _scan_b_body,
    grid=(HALF_STEPS // CHUNK,),
    in_specs=_COMMON_SPECS + [
        _full((B, MODEL_DIM)),
        _full((MODEL_DIM, MLP_DIM)),
        _full((1, MLP_DIM)),
        _full((MLP_DIM, MODEL_DIM)),
        _full((1, MODEL_DIM)),
    ],
    out_specs=_full((B, MODEL_DIM)),
    out_shape=jax.ShapeDtypeStruct((B, MODEL_DIM), jnp.float32),
    scratch_shapes=_SCRATCH,
)


@jax.jit
def kernel(tokens, lengths, embeddings, W_ih, b_ih, W_hh, b_hh, W0, b0,
           W1, b1):
    idx = jnp.flip(tokens, axis=1).T.reshape(-1)
    gather = _make_sc_gather()
    x0 = gather(idx[:HALF_ROWS], embeddings)
    x1 = gather(idx[HALF_ROWS:], embeddings)

    wih_t = W_ih.T
    whh_t = W_hh.T
    bias = (b_ih + b_hh)[None, :]
    act = jnp.broadcast_to((L - lengths)[:, None], (B, MODEL_DIM))
    act = act.astype(jnp.int32)
    w0_t = W0.T
    b0r = b0[None, :]
    w1p = jnp.zeros((MLP_DIM, MODEL_DIM), W1.dtype).at[:, :NUM_CLASSES].set(
        W1.T)
    b1p = jnp.zeros((1, MODEL_DIM), b1.dtype).at[0, :NUM_CLASSES].set(b1)
    m = (L - jnp.max(lengths)).astype(jnp.int32).reshape(1)

    h_mid = _scan_a_call(x0, wih_t, whh_t, bias, act, m)
    y = _scan_b_call(x1, wih_t, whh_t, bias, act, m, h_mid, w0_t, b0r, w1p,
                     b1p)
    return y[:, :NUM_CLASSES]


# speedup vs baseline: 106.5015x; 1.0108x over previous
import functools

import jax
import jax.numpy as jnp
from jax import lax
from jax.experimental import pallas as pl
from jax.experimental.pallas import tpu as pltpu
from jax.experimental.pallas import tpu_sc as plsc

MODEL_DIM = 128
MLP_DIM = 256
NUM_CLASSES = 3
B = 16
L = 2048
NUM_WORKERS = 32
GCHUNK = 128
BLK = 8
SEG0_STEPS = 512
SEG1_STEPS = L - SEG0_STEPS
CHUNK_A = 256
CHUNK_B = 512


def _gather_body(idx_hbm, table_hbm, out_hbm, idx_a, idx_b, rows_a, rows_b,
                 gsem_a, gsem_b, wsem_a, wsem_b, *, rows_per_worker):
    wid = lax.axis_index("s") * 2 + lax.axis_index("c")
    base = wid * rows_per_worker
    nchunk = rows_per_worker // GCHUNK
    idx = [idx_a, idx_b]
    rows = [rows_a, rows_b]
    gsem = [gsem_a, gsem_b]
    wsem = [wsem_a, wsem_b]
    gcopy = [None, None]
    wcopy = [None, None]

    pltpu.sync_copy(idx_hbm.at[pl.ds(base, GCHUNK)], idx_a)
    gcopy[0] = pltpu.async_copy(table_hbm.at[idx_a], rows_a, gsem_a)
    for c in range(nchunk):
        cur = c % 2
        nxt = 1 - cur
        if c + 1 < nchunk:
            off_n = base + (c + 1) * GCHUNK
            pltpu.sync_copy(idx_hbm.at[pl.ds(off_n, GCHUNK)], idx[nxt])
            if wcopy[nxt] is not None:
                wcopy[nxt].wait()
            gcopy[nxt] = pltpu.async_copy(table_hbm.at[idx[nxt]], rows[nxt],
                                          gsem[nxt])
        gcopy[cur].wait()
        wcopy[cur] = pltpu.async_copy(
            rows[cur], out_hbm.at[pl.ds(base + c * GCHUNK, GCHUNK)],
            wsem[cur])
    for w in wcopy:
        if w is not None:
            w.wait()


@functools.cache
def _make_sc_gather(n_rows):
    body = functools.partial(_gather_body,
                             rows_per_worker=n_rows // NUM_WORKERS)
    return pl.kernel(
        body,
        out_type=jax.ShapeDtypeStruct((n_rows, MODEL_DIM), jnp.float32),
        mesh=plsc.VectorSubcoreMesh(core_axis_name="c", subcore_axis_name="s",
                                    num_cores=2, num_subcores=16),
        scratch_types=[
            pltpu.VMEM((GCHUNK,), jnp.int32),
            pltpu.VMEM((GCHUNK,), jnp.int32),
            pltpu.VMEM((GCHUNK, MODEL_DIM), jnp.float32),
            pltpu.VMEM((GCHUNK, MODEL_DIM), jnp.float32),
            pltpu.SemaphoreType.DMA,
            pltpu.SemaphoreType.DMA,
            pltpu.SemaphoreType.DMA,
            pltpu.SemaphoreType.DMA,
        ],
    )


def _scan_chunk(x_ref, wih_ref, whh_ref, bias_ref, act_ref, m_ref, h_ref,
                pre_ref, t_base, chunk):
    @pl.when(m_ref[0] < t_base + chunk)
    def _active():
        pre_ref[...] = jnp.dot(
            x_ref[...], wih_ref[...],
            preferred_element_type=jnp.float32) + bias_ref[...]
        whh = whh_ref[...]
        act = act_ref[...]
        nblk0 = jnp.clip((m_ref[0] - t_base) // BLK, 0, chunk // BLK)

        def blk(kb, h):
            for jj in range(BLK):
                j = kb * BLK + jj
                x = pre_ref[pl.ds(j * B, B), :]
                h_new = jnp.tanh(
                    x + jnp.dot(h, whh, preferred_element_type=jnp.float32))
                h = jnp.where(act <= t_base + j, h_new, h)
            return h

        h_ref[...] = lax.fori_loop(nblk0, chunk // BLK, blk, h_ref[...])


def _scan_a_body(x_ref, wih_ref, whh_ref, bias_ref, act_ref, m_ref, out_ref,
                 h_ref, pre_ref):
    g = pl.program_id(0)
    ng = pl.num_programs(0)

    @pl.when(g == 0)
    def _init():
        h_ref[...] = jnp.zeros_like(h_ref)

    _scan_chunk(x_ref, wih_ref, whh_ref, bias_ref, act_ref, m_ref, h_ref,
                pre_ref, g * CHUNK_A, CHUNK_A)

    @pl.when(g == ng - 1)
    def _emit():
        out_ref[...] = h_ref[...]


def _scan_b_body(x_ref, wih_ref, whh_ref, bias_ref, act_ref, m_ref, h_in_ref,
                 w0_ref, b0_ref, w1_ref, b1_ref, out_ref, h_ref, pre_ref):
    g = pl.program_id(0)
    ng = pl.num_programs(0)

    @pl.when(g == 0)
    def _init():
        h_ref[...] = h_in_ref[...]

    _scan_chunk(x_ref, wih_ref, whh_ref, bias_ref, act_ref, m_ref, h_ref,
                pre_ref, SEG0_STEPS + g * CHUNK_B, CHUNK_B)

    @pl.when(g == ng - 1)
    def _mlp():
        h1 = jnp.maximum(
            jnp.dot(h_ref[...], w0_ref[...],
                    preferred_element_type=jnp.float32)
            + b0_ref[...], 0.0)
        h2 = jnp.maximum(
            jnp.dot(h1, w1_ref[...], preferred_element_type=jnp.float32)
            + b1_ref[...], 0.0)
        lane = lax.broadcasted_iota(jnp.int32, (B, MODEL_DIM), 1)
        logits = jnp.where(lane < NUM_CLASSES, h2, -1e30)
        m = jnp.max(logits, axis=-1, keepdims=True)
        s = logits - m
        out_ref[...] = s - jnp.log(jnp.sum(jnp.exp(s), axis=-1,
                                           keepdims=True))


def _full(shape):
    return pl.BlockSpec(shape, lambda g: tuple(0 for _ in shape))


def _common_specs(chunk):
    return [
        pl.BlockSpec((chunk * B, MODEL_DIM), lambda g: (g, 0)),
        _full((MODEL_DIM, MODEL_DIM)),
        _full((MODEL_DIM, MODEL_DIM)),
        _full((1, MODEL_DIM)),
        _full((B, MODEL_DIM)),
        pl.BlockSpec(memory_space=pltpu.SMEM),
    ]


def _scratch(chunk):
    return [pltpu.VMEM((B, MODEL_DIM), jnp.float32),
            pltpu.VMEM((chunk * B, MODEL_DIM), jnp.float32)]


_scan_a_call = pl.pallas_call(
    _scan_a_body,
    grid=(SEG0_STEPS // CHUNK_A,),
    in_specs=_common_specs(CHUNK_A),
    out_specs=_full((B, MODEL_DIM)),
    out_shape=jax.ShapeDtypeStruct((B, MODEL_DIM), jnp.float32),
    scratch_shapes=_scratch(CHUNK_A),
)

_scan_b_call = pl.pallas_call(
    _scan_b_body,
    grid=(SEG1_STEPS // CHUNK_B,),
    in_specs=_common_specs(CHUNK_B) + [
        _full((B, MODEL_DIM)),
        _full((MODEL_DIM, MLP_DIM)),
        _full((1, MLP_DIM)),
        _full((MLP_DIM, MODEL_DIM)),
        _full((1, MODEL_DIM)),
    ],
    out_specs=_full((B, MODEL_DIM)),
    out_shape=jax.ShapeDtypeStruct((B, MODEL_DIM), jnp.float32),
    scratch_shapes=_scratch(CHUNK_B),
)


@jax.jit
def kernel(tokens, lengths, embeddings, W_ih, b_ih, W_hh, b_hh, W0, b0,
           W1, b1):
    idx = jnp.flip(tokens, axis=1).T.reshape(-1)
    seg0 = SEG0_STEPS * B
    x0 = _make_sc_gather(seg0)(idx[:seg0], embeddings)
    x1 = _make_sc_gather(SEG1_STEPS * B)(idx[seg0:], embeddings)

    wih_t = W_ih.T
    whh_t = W_hh.T
    bias = (b_ih + b_hh)[None, :]
    act = jnp.broadcast_to((L - lengths)[:, None], (B, MODEL_DIM))
    act = act.astype(jnp.int32)
    w0_t = W0.T
    b0r = b0[None, :]
    w1p = jnp.zeros((MLP_DIM, MODEL_DIM), W1.dtype).at[:, :NUM_CLASSES].set(
        W1.T)
    b1p = jnp.zeros((1, MODEL_DIM), b1.dtype).at[0, :NUM_CLASSES].set(b1)
    m = (L - jnp.max(lengths)).astype(jnp.int32).reshape(1)

    h_mid = _scan_a_call(x0, wih_t, whh_t, bias, act, m)
    y = _scan_b_call(x1, wih_t, whh_t, bias, act, m, h_mid, w0_t, b0r, w1p,
                     b1p)
    return y[:, :NUM_CLASSES]
